# baseline (speedup 1.0000x reference)
import functools

import jax
import jax.numpy as jnp
from jax import lax
from jax.experimental import pallas as pl
from jax.experimental.pallas import tpu as pltpu, tpu_sc as plsc

N = 10000
H = 128
EPS = 1e-12

NC = 2
NS = 16
NW = NC * NS
K = 128
SEGW = 313
SEGP = 320

ROW_BLK = 1000


def _h0_body(x_ref, w_ref, b_ref, o_ref):
    y = lax.dot_general(x_ref[...], w_ref[...], (((1,), (1,)), ((), ())),
                        preferred_element_type=jnp.float32)
    y = y + b_ref[...]
    nrm = jnp.sqrt(jnp.sum(y * y, axis=1, keepdims=True))
    o_ref[...] = y / jnp.maximum(nrm, EPS)


def _combine_body(ppi_ref, res_ref, w_ref, b_ref, o_ref):
    y = lax.dot_general(ppi_ref[...], w_ref[...], (((1,), (1,)), ((), ())),
                        preferred_element_type=jnp.float32)
    o_ref[...] = jnp.maximum(y + b_ref[...], 0.0) + res_ref[...]


def _final_body(h_ref, w_ref, b_ref, o_ref):
    y = lax.dot_general(h_ref[...], w_ref[...], (((1,), (1,)), ((), ())),
                        preferred_element_type=jnp.float32)
    o_ref[...] = y + b_ref[...]


def _row_grid(n):
    return (n // ROW_BLK,)


def _tc_h0(x, w, b):
    return pl.pallas_call(
        _h0_body,
        grid=_row_grid(N),
        in_specs=[
            pl.BlockSpec((ROW_BLK, x.shape[1]), lambda i: (i, 0)),
            pl.BlockSpec(w.shape, lambda i: (0, 0)),
            pl.BlockSpec((1, H), lambda i: (0, 0)),
        ],
        out_specs=pl.BlockSpec((ROW_BLK, H), lambda i: (i, 0)),
        out_shape=jax.ShapeDtypeStruct((N, H), jnp.float32),
    )(x, w, b)


def _tc_combine(ppi, res, w, b):
    return pl.pallas_call(
        _combine_body,
        grid=_row_grid(N),
        in_specs=[
            pl.BlockSpec((ROW_BLK, H), lambda i: (i, 0)),
            pl.BlockSpec((ROW_BLK, H), lambda i: (i, 0)),
            pl.BlockSpec((H, H), lambda i: (0, 0)),
            pl.BlockSpec((1, H), lambda i: (0, 0)),
        ],
        out_specs=pl.BlockSpec((ROW_BLK, H), lambda i: (i, 0)),
        out_shape=jax.ShapeDtypeStruct((N, H), jnp.float32),
    )(ppi, res, w, b)


def _tc_final(h, w, b):
    l = w.shape[0]
    return pl.pallas_call(
        _final_body,
        grid=_row_grid(N),
        in_specs=[
            pl.BlockSpec((ROW_BLK, H), lambda i: (i, 0)),
            pl.BlockSpec((l, H), lambda i: (0, 0)),
            pl.BlockSpec((1, l), lambda i: (0, 0)),
        ],
        out_specs=pl.BlockSpec((ROW_BLK, l), lambda i: (i, 0)),
        out_shape=jax.ShapeDtypeStruct((N, l), jnp.float32),
    )(h, w, b)


def _sc_segment_sums(h, srcb, dstrb, wsb, wpb, starts, nbats, zeros):
    mesh = plsc.VectorSubcoreMesh(core_axis_name="c", subcore_axis_name="s",
                                  num_cores=NC, num_subcores=NS)

    @functools.partial(
        pl.kernel,
        mesh=mesh,
        compiler_params=pltpu.CompilerParams(needs_layout_passes=False),
        out_type=jax.ShapeDtypeStruct((2, NW, SEGP, H), jnp.float32),
        scratch_types=[
            pltpu.VMEM((SEGP, H), jnp.float32),
            pltpu.VMEM((SEGP, H), jnp.float32),
            pltpu.VMEM((K, H), jnp.float32),
            pltpu.VMEM((K, H), jnp.float32),
            pltpu.VMEM((8, K), jnp.int32),
            pltpu.VMEM((8, K), jnp.int32),
            pltpu.VMEM((8, K), jnp.float32),
            pltpu.VMEM((8, K), jnp.float32),
            pltpu.VMEM((32,), jnp.int32),
            pltpu.VMEM((32,), jnp.int32),
            pltpu.SemaphoreType.DMA,
            pltpu.SemaphoreType.DMA,
        ],
    )
    def sc_kernel(h_hbm, src_hbm, dstr_hbm, ws_hbm, wp_hbm,
                  starts_hbm, nbats_hbm, z_hbm, out_hbm,
                  acc_res, acc_ppi, rows0, rows1,
                  src_v, dstr_v, ws_v, wp_v, starts_v, nbats_v,
                  sem0, sem1):
        c = lax.axis_index("c")
        s = lax.axis_index("s")
        w = c * NS + s
        rows = (rows0, rows1)
        sems = (sem0, sem1)
        w16 = jnp.full((16,), w, jnp.int32)

        pltpu.sync_copy(starts_hbm, starts_v)
        pltpu.sync_copy(nbats_hbm, nbats_v)
        start = plsc.load_gather(starts_v, [w16])[0]
        nbat = plsc.load_gather(nbats_v, [w16])[0]

        for acc in (acc_res, acc_ppi):
            pltpu.sync_copy(z_hbm, acc.at[pl.ds(0, K)])
            pltpu.sync_copy(z_hbm, acc.at[pl.ds(K, K)])
            pltpu.sync_copy(z_hbm.at[pl.ds(0, SEGP - 2 * K)],
                            acc.at[pl.ds(2 * K, SEGP - 2 * K)])

        def fetch(b, par):
            base = pl.multiple_of(start + b * K, 8)
            pltpu.sync_copy(src_hbm.at[pl.ds(base, K)], src_v.at[par])
            pltpu.sync_copy(dstr_hbm.at[pl.ds(base, K)], dstr_v.at[par])
            pltpu.sync_copy(ws_hbm.at[pl.ds(base, K)], ws_v.at[par])
            pltpu.sync_copy(wp_hbm.at[pl.ds(base, K)], wp_v.at[par])
            pltpu.async_copy(h_hbm.at[src_v.at[par]], rows[par], sems[par])

        def process(b, par):
            pltpu.make_async_copy(h_hbm.at[src_v.at[par]], rows[par],
                                  sems[par]).wait()

            def group_body(g, carry):
                ws16 = ws_v[par, pl.ds(g * 16, 16)]
                wp16 = wp_v[par, pl.ds(g * 16, 16)]
                dr16 = dstr_v[par, pl.ds(g * 16, 16)]
                for j in range(16):
                    e = g * 16 + j
                    wsx = ws16[j]
                    wpx = wp16[j]
                    dre = dr16[j]
                    for ch in range(H // 16):
                        sl = pl.ds(ch * 16, 16)
                        r = rows[par][e, sl]
                        plsc.addupdate(acc_res.at[dre, sl], r * wsx)
                        plsc.addupdate(acc_ppi.at[dre, sl], r * wpx)
                return carry

            lax.fori_loop(0, K // 16, group_body, 0)

            @pl.when(b + 2 < nbat)
            def _():
                fetch(b + 2, par)

        @pl.when(nbat > 0)
        def _():
            fetch(0, 0)

        @pl.when(nbat > 1)
        def _():
            fetch(1, 1)

        def loop_body(i, carry):
            process(2 * i, 0)

            @pl.when(2 * i + 1 < nbat)
            def _():
                process(2 * i + 1, 1)

            return carry

        lax.fori_loop(0, (nbat + 1) // 2, loop_body, 0)

        pltpu.sync_copy(acc_res, out_hbm.at[0, w])
        pltpu.sync_copy(acc_ppi, out_hbm.at[1, w])

    return sc_kernel(h, srcb, dstrb, wsb, wpb, starts, nbats, zeros)


def _bucket_edges(src, dst, w_self, w_ppi):
    e = src.shape[0]
    cap = e + NW * K
    seg = dst // SEGW
    order = jnp.argsort(seg, stable=True)
    seg_s = seg[order]
    counts = jnp.bincount(seg, length=NW).astype(jnp.int32)
    nbats = (counts + K - 1) // K
    starts_p = jnp.concatenate(
        [jnp.zeros((1,), jnp.int32), jnp.cumsum(nbats * K).astype(jnp.int32)])
    starts_u = jnp.concatenate(
        [jnp.zeros((1,), jnp.int32), jnp.cumsum(counts).astype(jnp.int32)])
    pos = starts_p[seg_s] + (jnp.arange(e, dtype=jnp.int32) - starts_u[seg_s])

    def fill(vals, init):
        return jnp.full((cap,), init, vals.dtype).at[pos].set(vals[order])

    srcb = fill(src, 0)
    dstrb = fill(dst - seg * SEGW, 0)
    wsb = fill(w_self, 0.0)
    wpb = fill(w_ppi, 0.0)
    return srcb, dstrb, wsb, wpb, starts_p[:NW], nbats


def kernel(inputs, edge_index, edge_ppi, edge_self, W_in, b_in, input_bias,
           W_ppi1, b_ppi1, W_ppi2, b_ppi2, W_out, b_out):
    srcb, dstrb, wsb, wpb, starts, nbats = _bucket_edges(
        edge_index[0], edge_index[1], edge_self, edge_ppi)
    zeros = jnp.zeros((K, H), jnp.float32)

    bias0 = (b_in + input_bias).reshape(1, H)
    h = _tc_h0(inputs, W_in, bias0)

    for w, b in ((W_ppi1, b_ppi1), (W_ppi2, b_ppi2)):
        sums = _sc_segment_sums(h, srcb, dstrb, wsb, wpb, starts, nbats,
                                zeros)
        res = sums[0][:, :SEGW].reshape(NW * SEGW, H)[:N]
        ppi = sums[1][:, :SEGW].reshape(NW * SEGW, H)[:N]
        h = _tc_combine(ppi, res, w, b.reshape(1, H))

    return _tc_final(h, W_out, b_out.reshape(1, W_out.shape[0]))


# speedup vs baseline: 4.3307x; 4.3307x over previous
import functools

import jax
import jax.numpy as jnp
from jax import lax
from jax.experimental import pallas as pl
from jax.experimental.pallas import tpu as pltpu, tpu_sc as plsc

N = 10000
H = 128
EPS = 1e-12

NC = 2
NS = 16
K = 128

ROW_BLK = 1000


def _h0_body(x_ref, w_ref, b_ref, o_ref):
    y = lax.dot_general(x_ref[...], w_ref[...], (((1,), (1,)), ((), ())),
                        preferred_element_type=jnp.float32)
    y = y + b_ref[...]
    nrm = jnp.sqrt(jnp.sum(y * y, axis=1, keepdims=True))
    o_ref[...] = y / jnp.maximum(nrm, EPS)


def _combine_body(ppi_ref, res_ref, w_ref, b_ref, o_ref):
    y = lax.dot_general(ppi_ref[...], w_ref[...], (((1,), (1,)), ((), ())),
                        preferred_element_type=jnp.float32)
    o_ref[...] = jnp.maximum(y + b_ref[...], 0.0) + res_ref[...]


def _final_body(h_ref, w_ref, b_ref, o_ref):
    y = lax.dot_general(h_ref[...], w_ref[...], (((1,), (1,)), ((), ())),
                        preferred_element_type=jnp.float32)
    o_ref[...] = y + b_ref[...]


def _row_grid(n):
    return (n // ROW_BLK,)


def _tc_h0(x, w, b):
    return pl.pallas_call(
        _h0_body,
        grid=_row_grid(N),
        in_specs=[
            pl.BlockSpec((ROW_BLK, x.shape[1]), lambda i: (i, 0)),
            pl.BlockSpec(w.shape, lambda i: (0, 0)),
            pl.BlockSpec((1, H), lambda i: (0, 0)),
        ],
        out_specs=pl.BlockSpec((ROW_BLK, H), lambda i: (i, 0)),
        out_shape=jax.ShapeDtypeStruct((N, H), jnp.float32),
    )(x, w, b)


def _tc_combine(ppi, res, w, b):
    return pl.pallas_call(
        _combine_body,
        grid=_row_grid(N),
        in_specs=[
            pl.BlockSpec((ROW_BLK, H), lambda i: (i, 0)),
            pl.BlockSpec((ROW_BLK, H), lambda i: (i, 0)),
            pl.BlockSpec((H, H), lambda i: (0, 0)),
            pl.BlockSpec((1, H), lambda i: (0, 0)),
        ],
        out_specs=pl.BlockSpec((ROW_BLK, H), lambda i: (i, 0)),
        out_shape=jax.ShapeDtypeStruct((N, H), jnp.float32),
    )(ppi, res, w, b)


def _tc_final(h, w, b):
    l = w.shape[0]
    return pl.pallas_call(
        _final_body,
        grid=_row_grid(N),
        in_specs=[
            pl.BlockSpec((ROW_BLK, H), lambda i: (i, 0)),
            pl.BlockSpec((l, H), lambda i: (0, 0)),
            pl.BlockSpec((1, l), lambda i: (0, 0)),
        ],
        out_specs=pl.BlockSpec((ROW_BLK, l), lambda i: (i, 0)),
        out_shape=jax.ShapeDtypeStruct((N, l), jnp.float32),
    )(h, w, b)


def _sc_segment_sums(h, src, dst, w2, zeros, e_pad):
    ept = e_pad // NS
    nb = ept // K
    row_stride, row_span = 624, 640

    mesh = plsc.VectorSubcoreMesh(core_axis_name="c", subcore_axis_name="s",
                                  num_cores=NC, num_subcores=NS)

    @functools.partial(
        pl.kernel,
        mesh=mesh,
        out_type=jax.ShapeDtypeStruct((NC, N, H), jnp.float32),
        scratch_types=[
            pltpu.VMEM_SHARED((N, H), jnp.float32),
            pltpu.VMEM((K,), jnp.int32),
            pltpu.VMEM((K,), jnp.int32),
            pltpu.VMEM((K,), jnp.float32),
            pltpu.VMEM((K, H), jnp.float32),
            pltpu.VMEM((K, H), jnp.float32),
            pltpu.SemaphoreType.DMA,
            pltpu.SemaphoreType.DMA,
        ],
    )
    def sc_kernel(h_hbm, src_hbm, dst_hbm, w2_hbm, z_hbm, out_hbm,
                  acc, src_v, dst_v, w_v, rows_v, prod_v, sem, semb):
        c = lax.axis_index("c")
        s = lax.axis_index("s")
        hk = K // 2

        pltpu.sync_copy(z_hbm.at[pl.ds(0, K)], rows_v)
        for z in range(row_span // K):
            pltpu.sync_copy(rows_v,
                            acc.at[pl.ds(s * row_stride + z * K, K)])
        plsc.subcore_barrier()

        def batch_body(b, carry):
            base = s * ept + b * K
            pltpu.sync_copy(src_hbm.at[pl.ds(base, K)], src_v)
            pltpu.sync_copy(dst_hbm.at[pl.ds(base, K)], dst_v)
            pltpu.sync_copy(w2_hbm.at[c, pl.ds(base, K)], w_v)
            cp_a = pltpu.async_copy(h_hbm.at[src_v.at[pl.ds(0, hk)]],
                                    rows_v.at[pl.ds(0, hk)], sem)
            cp_b = pltpu.async_copy(h_hbm.at[src_v.at[pl.ds(hk, hk)]],
                                    rows_v.at[pl.ds(hk, hk)], semb)
            cp_a.wait()
            cp_b.wait()

            def group_body(g, carry2):
                w16 = w_v[pl.ds(g * 16, 16)]
                for j in range(16):
                    e = g * 16 + j
                    wb = w16[j]
                    for ch in range(H // 16):
                        sl = pl.ds(ch * 16, 16)
                        prod_v[e, sl] = rows_v[e, sl] * wb
                return carry2

            lax.fori_loop(0, K // 16, group_body, 0)
            pltpu.sync_copy(prod_v, acc.at[dst_v], add=True)
            return carry

        lax.fori_loop(0, nb, batch_body, 0)
        plsc.subcore_barrier()

        pltpu.sync_copy(acc.at[pl.ds(s * row_stride, row_span)],
                        out_hbm.at[c, pl.ds(s * row_stride, row_span)])

    return sc_kernel(h, src, dst, w2, zeros)


def kernel(inputs, edge_index, edge_ppi, edge_self, W_in, b_in, input_bias,
           W_ppi1, b_ppi1, W_ppi2, b_ppi2, W_out, b_out):
    e = edge_index.shape[1]
    e_pad = ((e + NS * K - 1) // (NS * K)) * (NS * K)
    pad = e_pad - e

    src = jnp.concatenate([edge_index[0], jnp.zeros((pad,), jnp.int32)])
    dst = jnp.concatenate([edge_index[1], jnp.zeros((pad,), jnp.int32)])
    wpad = jnp.zeros((pad,), jnp.float32)
    w2 = jnp.stack([jnp.concatenate([edge_self, wpad]),
                    jnp.concatenate([edge_ppi, wpad])])
    zeros = jnp.zeros((K, H), jnp.float32)

    bias0 = (b_in + input_bias).reshape(1, H)
    h = _tc_h0(inputs, W_in, bias0)

    for w, b in ((W_ppi1, b_ppi1), (W_ppi2, b_ppi2)):
        sums = _sc_segment_sums(h, src, dst, w2, zeros, e_pad)
        h = _tc_combine(sums[1], sums[0], w, b.reshape(1, H))

    return _tc_final(h, W_out, b_out.reshape(1, W_out.shape[0]))
